# baseline (speedup 1.0000x reference)
import jax
import jax.numpy as jnp
from jax import lax
from jax.experimental import pallas as pl
from jax.experimental.pallas import tpu as pltpu
from jax.experimental.pallas import tpu_sc as plsc

_B = 8
_C = 192
_HW = 224
_N = _HW * _HW
_OFF = 64
_K = 2
_LANES = 16
_VECS = _N // _LANES


_ROWS = _N // 128


def _top2_body(flat_hbm, vals_hbm, idxs_hbm, buf, vbuf, ibuf):
    nc = 2
    wid = lax.axis_index("s") * nc + lax.axis_index("c")

    @pl.when(wid < _B)
    def _():
        pltpu.sync_copy(flat_hbm.at[wid], buf)
        lanes = lax.iota(jnp.int32, 16)
        ninf = jnp.full((16,), -jnp.inf, jnp.float32)
        zeros = jnp.zeros((16,), jnp.int32)

        def step(k, carry):
            m1, i1, m2, i2 = carry
            r = k // 14
            lg = k - r * 14
            v = buf[r, pl.ds(lg * _LANES, _LANES)]
            idx = lanes + k * _LANES
            gt1 = v > m1
            gt2 = v > m2
            m2n = jnp.where(gt1, m1, jnp.where(gt2, v, m2))
            i2n = jnp.where(gt1, i1, jnp.where(gt2, idx, i2))
            m1n = jnp.where(gt1, v, m1)
            i1n = jnp.where(gt1, idx, i1)
            return m1n, i1n, m2n, i2n

        m1, i1, m2, i2 = lax.fori_loop(
            0, _VECS, step, (ninf, zeros, ninf, zeros))

        vbuf[pl.ds(0, _LANES)] = m1
        vbuf[pl.ds(_LANES, _LANES)] = m2
        ibuf[pl.ds(0, _LANES)] = i1
        ibuf[pl.ds(_LANES, _LANES)] = i2
        pltpu.sync_copy(vbuf, vals_hbm.at[wid])
        pltpu.sync_copy(ibuf, idxs_hbm.at[wid])


def _sc_top2(flat):
    mesh = plsc.VectorSubcoreMesh(core_axis_name="c", subcore_axis_name="s")
    return pl.kernel(
        _top2_body,
        out_type=(
            jax.ShapeDtypeStruct((_B, 128), jnp.float32),
            jax.ShapeDtypeStruct((_B, 128), jnp.int32),
        ),
        mesh=mesh,
        scratch_types=[
            pltpu.VMEM((_HW, _HW), jnp.float32),
            pltpu.VMEM((128,), jnp.float32),
            pltpu.VMEM((128,), jnp.int32),
        ],
        compiler_params=pltpu.CompilerParams(use_tc_tiling_on_sc=True),
    )(flat)


def _merge_top2(vals_ref, idxs_ref, b):
    v1 = jnp.float32(-jnp.inf)
    v2 = jnp.float32(-jnp.inf)
    big = jnp.int32(2**31 - 1)
    i1 = big
    i2 = big
    for j in range(_K):
        for l in range(_LANES):
            v = vals_ref[b, j * _LANES + l]
            i = idxs_ref[b, j * _LANES + l]
            b1 = (v > v1) | ((v == v1) & (i < i1))
            b2 = (v > v2) | ((v == v2) & (i < i2))
            nv2 = jnp.where(b1, v1, jnp.where(b2, v, v2))
            ni2 = jnp.where(b1, i1, jnp.where(b2, i, i2))
            v1 = jnp.where(b1, v, v1)
            i1 = jnp.where(b1, i, i1)
            v2, i2 = nv2, ni2
    lo = jnp.minimum(i1, i2)
    hi = jnp.maximum(i1, i2)
    return lo, hi


_CB = 32
_NCB = _C // _CB
_WR = _OFF + 8
_WC = 384
_TSTEPS = _B * _K * _NCB


_QC = _CB // 4


def _win_dma(target_ref, win, sems, b, x, cblk, slot, do_start):
    xa = (x // 8) * 8
    for h in range(4):
        cpy = pltpu.make_async_copy(
            target_ref.at[b, pl.ds(cblk * _CB + h * _QC, _QC),
                          pl.ds(xa, _WR), :],
            win.at[slot, pl.ds(h * _QC, _QC)],
            sems.at[slot],
        )
        if do_start:
            cpy.start()
        else:
            cpy.wait()


def _gather_body(vals_ref, idxs_ref, target_ref, out_ref, xy_ref, win, sems):
    i = pl.program_id(0)
    c = pl.program_id(1)
    t = i * _NCB + c
    b = i // _K
    s = i - b * _K

    @pl.when(t == 0)
    def _():
        for bb in range(_B):
            lo, hi = _merge_top2(vals_ref, idxs_ref, bb)
            for ss, f in ((0, lo), (1, hi)):
                fx = f // _HW
                xy_ref[bb, ss, 0] = fx
                xy_ref[bb, ss, 1] = f - fx * _HW
        _win_dma(target_ref, win, sems, 0, xy_ref[0, 0, 0], 0, 0, True)

    @pl.when(t + 1 < _TSTEPS)
    def _():
        tn = t + 1
        inx = tn // _NCB
        cn = tn - inx * _NCB
        bn = inx // _K
        sn = inx - bn * _K
        _win_dma(target_ref, win, sems, bn,
                 xy_ref[bn, sn, 0], cn, tn % 2, True)

    x = xy_ref[b, s, 0]
    y = xy_ref[b, s, 1]
    slot = t % 2
    _win_dma(target_ref, win, sems, b, x, c, slot, False)
    dx = x - (x // 8) * 8
    a = pltpu.roll(win[slot], _WC - y, axis=2)[:, :, :_OFF]
    r = pltpu.roll(a, _WR - dx, axis=1)
    out_ref[0] = r[:, :_OFF, :]


def _tc_gather(vals, idxs, target):
    return pl.pallas_call(
        _gather_body,
        grid=(_B * _K, _NCB),
        in_specs=[
            pl.BlockSpec(memory_space=pltpu.SMEM),
            pl.BlockSpec(memory_space=pltpu.SMEM),
            pl.BlockSpec(memory_space=pl.ANY),
        ],
        out_specs=pl.BlockSpec(
            (1, _CB, _OFF, _OFF), lambda i, c: (i, c, 0, 0)),
        out_shape=jax.ShapeDtypeStruct((_B * _K, _C, _OFF, _OFF), jnp.float32),
        scratch_shapes=[
            pltpu.SMEM((_B, _K, 2), jnp.int32),
            pltpu.VMEM((2, _CB, _WR, _WC), jnp.float32),
            pltpu.SemaphoreType.DMA((2,)),
        ],
    )(vals, idxs, target)


def kernel(target, logits):
    flat = logits.reshape(_B, _HW, _HW)
    vals, idxs = _sc_top2(flat)
    out = _tc_gather(vals, idxs, target)
    return out.reshape(_B, _K, _C, _OFF, _OFF)


# speedup vs baseline: 1.4531x; 1.4531x over previous
import jax
import jax.numpy as jnp
from jax import lax
from jax.experimental import pallas as pl
from jax.experimental.pallas import tpu as pltpu
from jax.experimental.pallas import tpu_sc as plsc

_B = 8
_C = 192
_HW = 224
_N = _HW * _HW
_OFF = 64
_K = 2
_LANES = 16
_VECS = _N // _LANES


_ROWS = _N // 128


def _top2_body(flat_hbm, vals_hbm, idxs_hbm, buf, vbuf, ibuf):
    nc = 2
    wid = lax.axis_index("s") * nc + lax.axis_index("c")

    @pl.when(wid < _B)
    def _():
        pltpu.sync_copy(flat_hbm.at[wid], buf)
        lanes = lax.iota(jnp.int32, 16)
        ninf = jnp.full((16,), -jnp.inf, jnp.float32)
        zeros = jnp.zeros((16,), jnp.int32)

        def step(k, carry):
            m1, i1, m2, i2 = carry
            r = k // 14
            lg = k - r * 14
            v = buf[r, pl.ds(lg * _LANES, _LANES)]
            idx = lanes + k * _LANES
            gt1 = v > m1
            gt2 = v > m2
            m2n = jnp.where(gt1, m1, jnp.where(gt2, v, m2))
            i2n = jnp.where(gt1, i1, jnp.where(gt2, idx, i2))
            m1n = jnp.where(gt1, v, m1)
            i1n = jnp.where(gt1, idx, i1)
            return m1n, i1n, m2n, i2n

        m1, i1, m2, i2 = lax.fori_loop(
            0, _VECS, step, (ninf, zeros, ninf, zeros))

        vbuf[pl.ds(0, _LANES)] = m1
        vbuf[pl.ds(_LANES, _LANES)] = m2
        ibuf[pl.ds(0, _LANES)] = i1
        ibuf[pl.ds(_LANES, _LANES)] = i2
        pltpu.sync_copy(vbuf, vals_hbm.at[wid])
        pltpu.sync_copy(ibuf, idxs_hbm.at[wid])


def _sc_top2(flat):
    mesh = plsc.VectorSubcoreMesh(core_axis_name="c", subcore_axis_name="s")
    return pl.kernel(
        _top2_body,
        out_type=(
            jax.ShapeDtypeStruct((_B, 128), jnp.float32),
            jax.ShapeDtypeStruct((_B, 128), jnp.int32),
        ),
        mesh=mesh,
        scratch_types=[
            pltpu.VMEM((_HW, _HW), jnp.float32),
            pltpu.VMEM((128,), jnp.float32),
            pltpu.VMEM((128,), jnp.int32),
        ],
        compiler_params=pltpu.CompilerParams(use_tc_tiling_on_sc=True),
    )(flat)


def _merge_top2(vals_ref, idxs_ref, b):
    v1 = jnp.float32(-jnp.inf)
    v2 = jnp.float32(-jnp.inf)
    big = jnp.int32(2**31 - 1)
    i1 = big
    i2 = big
    for j in range(_K):
        for l in range(_LANES):
            v = vals_ref[b, j * _LANES + l]
            i = idxs_ref[b, j * _LANES + l]
            b1 = (v > v1) | ((v == v1) & (i < i1))
            b2 = (v > v2) | ((v == v2) & (i < i2))
            nv2 = jnp.where(b1, v1, jnp.where(b2, v, v2))
            ni2 = jnp.where(b1, i1, jnp.where(b2, i, i2))
            v1 = jnp.where(b1, v, v1)
            i1 = jnp.where(b1, i, i1)
            v2, i2 = nv2, ni2
    lo = jnp.minimum(i1, i2)
    hi = jnp.maximum(i1, i2)
    return lo, hi


_CB = 32
_NCB = _C // _CB
_WR = _OFF + 8
_WC = 256
_TSTEPS = _B * _K * _NCB
_HC = _CB // 2
_NSLOT = 4


def _win_dma(target_ref, win, sems, b, x, y, cblk, slot, do_start):
    xa = (x // 8) * 8
    ya = (y // 128) * 128
    narrow = y - ya <= 128 - _OFF

    def issue(width):
        for h in range(2):
            cpy = pltpu.make_async_copy(
                target_ref.at[b, pl.ds(cblk * _CB + h * _HC, _HC),
                              pl.ds(xa, _WR), pl.ds(ya, width)],
                win.at[slot, pl.ds(h * _HC, _HC), :, pl.ds(0, width)],
                sems.at[slot],
            )
            if do_start:
                cpy.start()
            else:
                cpy.wait()

    @pl.when(narrow)
    def _():
        issue(128)

    @pl.when(jnp.logical_not(narrow))
    def _():
        issue(_WC)


def _gather_body(vals_ref, idxs_ref, target_ref, out_ref, xy_ref, win, sems):
    i = pl.program_id(0)
    c = pl.program_id(1)
    t = i * _NCB + c
    b = i // _K
    s = i - b * _K

    @pl.when(t == 0)
    def _():
        for bb in range(_B):
            lo, hi = _merge_top2(vals_ref, idxs_ref, bb)
            for ss, f in ((0, lo), (1, hi)):
                fx = f // _HW
                xy_ref[bb, ss, 0] = fx
                xy_ref[bb, ss, 1] = f - fx * _HW
        for tp in range(_NSLOT - 1):
            _win_dma(target_ref, win, sems, 0,
                     xy_ref[0, 0, 0], xy_ref[0, 0, 1], tp, tp, True)

    @pl.when(t + _NSLOT - 1 < _TSTEPS)
    def _():
        tn = t + _NSLOT - 1
        inx = tn // _NCB
        cn = tn - inx * _NCB
        bn = inx // _K
        sn = inx - bn * _K
        _win_dma(target_ref, win, sems, bn,
                 xy_ref[bn, sn, 0], xy_ref[bn, sn, 1], cn,
                 tn % _NSLOT, True)

    x = xy_ref[b, s, 0]
    y = xy_ref[b, s, 1]
    slot = t % _NSLOT
    _win_dma(target_ref, win, sems, b, x, y, c, slot, False)
    dx = x - (x // 8) * 8
    dy = y - (y // 128) * 128
    a = pltpu.roll(win[slot], _WC - dy, axis=2)[:, :, :_OFF]
    r = pltpu.roll(a, _WR - dx, axis=1)[:, :_OFF, :]
    out_ref[0] = r


def _tc_gather(vals, idxs, target):
    return pl.pallas_call(
        _gather_body,
        grid=(_B * _K, _NCB),
        in_specs=[
            pl.BlockSpec(memory_space=pltpu.SMEM),
            pl.BlockSpec(memory_space=pltpu.SMEM),
            pl.BlockSpec(memory_space=pl.ANY),
        ],
        out_specs=pl.BlockSpec(
            (1, _CB, _OFF, _OFF), lambda i, c: (i, c, 0, 0)),
        out_shape=jax.ShapeDtypeStruct((_B * _K, _C, _OFF, _OFF),
                                       jnp.float32),
        scratch_shapes=[
            pltpu.SMEM((_B, _K, 2), jnp.int32),
            pltpu.VMEM((_NSLOT, _CB, _WR, _WC), jnp.float32),
            pltpu.SemaphoreType.DMA((_NSLOT,)),
        ],
    )(vals, idxs, target)


def kernel(target, logits):
    flat = logits.reshape(_B, _HW, _HW)
    vals, idxs = _sc_top2(flat)
    out = _tc_gather(vals, idxs, target)
    return out.reshape(_B, _K, _C, _OFF, _OFF)


# speedup vs baseline: 1.4676x; 1.0100x over previous
import jax
import jax.numpy as jnp
from jax import lax
from jax.experimental import pallas as pl
from jax.experimental.pallas import tpu as pltpu
from jax.experimental.pallas import tpu_sc as plsc

_B = 8
_C = 192
_HW = 224
_N = _HW * _HW
_OFF = 64
_K = 2
_LANES = 16
_VECS = _N // _LANES


_ROWS = _N // 128


def _top2_body(flat_hbm, vals_hbm, idxs_hbm, buf, vbuf, ibuf):
    nc = 2
    wid = lax.axis_index("s") * nc + lax.axis_index("c")

    @pl.when(wid < _B)
    def _():
        pltpu.sync_copy(flat_hbm.at[wid], buf)
        lanes = lax.iota(jnp.int32, 16)
        ninf = jnp.full((16,), -jnp.inf, jnp.float32)
        zeros = jnp.zeros((16,), jnp.int32)

        def step(k, carry):
            m1, i1, m2, i2 = carry
            r = k // 14
            lg = k - r * 14
            v = buf[r, pl.ds(lg * _LANES, _LANES)]
            idx = lanes + k * _LANES
            gt1 = v > m1
            gt2 = v > m2
            m2n = jnp.where(gt1, m1, jnp.where(gt2, v, m2))
            i2n = jnp.where(gt1, i1, jnp.where(gt2, idx, i2))
            m1n = jnp.where(gt1, v, m1)
            i1n = jnp.where(gt1, idx, i1)
            return m1n, i1n, m2n, i2n

        m1, i1, m2, i2 = lax.fori_loop(
            0, _VECS, step, (ninf, zeros, ninf, zeros))

        vbuf[pl.ds(0, _LANES)] = m1
        vbuf[pl.ds(_LANES, _LANES)] = m2
        ibuf[pl.ds(0, _LANES)] = i1
        ibuf[pl.ds(_LANES, _LANES)] = i2
        pltpu.sync_copy(vbuf, vals_hbm.at[wid])
        pltpu.sync_copy(ibuf, idxs_hbm.at[wid])


def _sc_top2(flat):
    mesh = plsc.VectorSubcoreMesh(core_axis_name="c", subcore_axis_name="s")
    return pl.kernel(
        _top2_body,
        out_type=(
            jax.ShapeDtypeStruct((_B, 128), jnp.float32),
            jax.ShapeDtypeStruct((_B, 128), jnp.int32),
        ),
        mesh=mesh,
        scratch_types=[
            pltpu.VMEM((_HW, _HW), jnp.float32),
            pltpu.VMEM((128,), jnp.float32),
            pltpu.VMEM((128,), jnp.int32),
        ],
        compiler_params=pltpu.CompilerParams(use_tc_tiling_on_sc=True),
    )(flat)


def _merge_top2(vals_ref, idxs_ref, b):
    v1 = jnp.float32(-jnp.inf)
    v2 = jnp.float32(-jnp.inf)
    big = jnp.int32(2**31 - 1)
    i1 = big
    i2 = big
    for j in range(_K):
        for l in range(_LANES):
            v = vals_ref[b, j * _LANES + l]
            i = idxs_ref[b, j * _LANES + l]
            b1 = (v > v1) | ((v == v1) & (i < i1))
            b2 = (v > v2) | ((v == v2) & (i < i2))
            nv2 = jnp.where(b1, v1, jnp.where(b2, v, v2))
            ni2 = jnp.where(b1, i1, jnp.where(b2, i, i2))
            v1 = jnp.where(b1, v, v1)
            i1 = jnp.where(b1, i, i1)
            v2, i2 = nv2, ni2
    lo = jnp.minimum(i1, i2)
    hi = jnp.maximum(i1, i2)
    return lo, hi


_CB = 32
_NCB = _C // _CB
_WR = _OFF + 8
_WC = 256
_TSTEPS = _B * _K * _NCB
_HC = _CB // 2
_NSLOT = 6


def _win_dma(target_ref, win, sems, b, x, y, cblk, slot, do_start):
    xa = (x // 8) * 8
    ya = (y // 128) * 128
    narrow = y - ya <= 128 - _OFF

    def issue(width):
        for h in range(2):
            cpy = pltpu.make_async_copy(
                target_ref.at[b, pl.ds(cblk * _CB + h * _HC, _HC),
                              pl.ds(xa, _WR), pl.ds(ya, width)],
                win.at[slot, pl.ds(h * _HC, _HC), :, pl.ds(0, width)],
                sems.at[slot],
            )
            if do_start:
                cpy.start()
            else:
                cpy.wait()

    @pl.when(narrow)
    def _():
        issue(128)

    @pl.when(jnp.logical_not(narrow))
    def _():
        issue(_WC)


def _gather_body(vals_ref, idxs_ref, target_ref, out_ref, xy_ref, win, sems):
    i = pl.program_id(0)
    c = pl.program_id(1)
    t = i * _NCB + c
    b = i // _K
    s = i - b * _K

    @pl.when(t == 0)
    def _():
        for bb in range(_B):
            lo, hi = _merge_top2(vals_ref, idxs_ref, bb)
            for ss, f in ((0, lo), (1, hi)):
                fx = f // _HW
                xy_ref[bb, ss, 0] = fx
                xy_ref[bb, ss, 1] = f - fx * _HW
        for tp in range(_NSLOT - 1):
            _win_dma(target_ref, win, sems, 0,
                     xy_ref[0, 0, 0], xy_ref[0, 0, 1], tp, tp, True)

    @pl.when(t + _NSLOT - 1 < _TSTEPS)
    def _():
        tn = t + _NSLOT - 1
        inx = tn // _NCB
        cn = tn - inx * _NCB
        bn = inx // _K
        sn = inx - bn * _K
        _win_dma(target_ref, win, sems, bn,
                 xy_ref[bn, sn, 0], xy_ref[bn, sn, 1], cn,
                 tn % _NSLOT, True)

    x = xy_ref[b, s, 0]
    y = xy_ref[b, s, 1]
    slot = t % _NSLOT
    _win_dma(target_ref, win, sems, b, x, y, c, slot, False)
    dx = x - (x // 8) * 8
    dy = y - (y // 128) * 128
    a = pltpu.roll(win[slot], _WC - dy, axis=2)[:, :, :_OFF]
    r = pltpu.roll(a, _WR - dx, axis=1)[:, :_OFF, :]
    out_ref[0] = r


def _tc_gather(vals, idxs, target):
    return pl.pallas_call(
        _gather_body,
        grid=(_B * _K, _NCB),
        in_specs=[
            pl.BlockSpec(memory_space=pltpu.SMEM),
            pl.BlockSpec(memory_space=pltpu.SMEM),
            pl.BlockSpec(memory_space=pl.ANY),
        ],
        out_specs=pl.BlockSpec(
            (1, _CB, _OFF, _OFF), lambda i, c: (i, c, 0, 0)),
        out_shape=jax.ShapeDtypeStruct((_B * _K, _C, _OFF, _OFF),
                                       jnp.float32),
        scratch_shapes=[
            pltpu.SMEM((_B, _K, 2), jnp.int32),
            pltpu.VMEM((_NSLOT, _CB, _WR, _WC), jnp.float32),
            pltpu.SemaphoreType.DMA((_NSLOT,)),
        ],
    )(vals, idxs, target)


def kernel(target, logits):
    flat = logits.reshape(_B, _HW, _HW)
    vals, idxs = _sc_top2(flat)
    out = _tc_gather(vals, idxs, target)
    return out.reshape(_B, _K, _C, _OFF, _OFF)
